# Initial kernel scaffold; baseline (speedup 1.0000x reference)
import jax
import jax.numpy as jnp
from jax.experimental import pallas as pl


def kernel(x, xids, wids, lora_A, lora_B):
    raise NotImplementedError("write your pallas kernel here")


# speedup vs baseline: 2.2027x; 2.2027x over previous
import functools

import jax
import jax.numpy as jnp
from jax import lax
from jax.experimental import pallas as pl
from jax.experimental.pallas import tpu as pltpu
from jax.experimental.pallas import tpu_sc as plsc

N_BATCH = 256
L_ROWS = 512
N_W = 128
D_IN = 4096
D_OUT = 4096
R = 16
LANES = 16
N_WORKERS = 32
ROWS_PER_WORKER = (N_BATCH * N_W) // N_WORKERS
UFLAT_PER_WORKER = ROWS_PER_WORKER * R


def _mm1_body(x_ref, a_ref, v_ref):
    v_ref[...] = jnp.dot(
        x_ref[...], a_ref[...], preferred_element_type=jnp.float32
    )


def _mm2_body(u_ref, b_ref, z_ref, uf16_ref):
    @pl.when(pl.program_id(0) == 0)
    def _():
        uf16_ref[...] = u_ref[...].astype(jnp.bfloat16)

    acc = jnp.dot(uf16_ref[...], b_ref[...], preferred_element_type=jnp.float32)
    z_ref[...] = acc * 2.0


def _sc_combine_body(xids_hbm, wids_hbm, v_hbm, out_hbm,
                     xv_ref, wv_ref, rows_ref, u_ref, sem):
    w = lax.axis_index("s") * 2 + lax.axis_index("c")
    iota = lax.iota(jnp.int32, LANES)

    def _zero(i, _):
        for u in range(8):
            u_ref[pl.ds((i * 8 + u) * LANES, LANES)] = jnp.zeros(
                (LANES,), jnp.float32
            )
        return 0

    lax.fori_loop(0, UFLAT_PER_WORKER // (LANES * 8), _zero, 0)

    def process_group(l_base, nvalid):
        pltpu.sync_copy(xids_hbm.at[pl.ds(l_base, LANES)], xv_ref)
        pltpu.sync_copy(wids_hbm.at[pl.ds(l_base, LANES)], wv_ref)
        xv = xv_ref[...]
        wv = wv_ref[...]
        lv = l_base + iota
        sv = jnp.where(
            lv < 128,
            lv,
            jnp.where(
                lv < 256,
                128 + ((lv - 128) >> 1),
                192 + ((lv - 256) >> 2),
            ),
        )
        pltpu.async_copy(v_hbm.at[xv], rows_ref, sem).wait()
        rloc = sv * N_W + wv - w * ROWS_PER_WORKER
        for j in range(LANES):
            jfull = jnp.full((LANES,), j, jnp.int32)
            mask = jfull < nvalid
            lane_j = iota == j
            wj = jnp.max(jnp.where(lane_j, wv, 0))
            rlj = jnp.max(jnp.where(lane_j, rloc, 0))
            offs = rlj * R + iota
            offs = jnp.where(mask, offs, 0)
            row = rows_ref[j, pl.ds(wj * R, R)]
            plsc.addupdate_scatter(u_ref, [offs], row, mask=mask)

    l1 = jnp.where(
        w < 16, 8 * w, jnp.where(w < 24, 16 * w - 128, 32 * w - 512)
    )
    n1 = jnp.where(w < 16, 8, 16)
    process_group(l1, n1)

    @pl.when(w >= 24)
    def _():
        process_group(32 * w - 512 + 16, jnp.int32(16))

    pltpu.sync_copy(
        u_ref, out_hbm.at[pl.ds(w * UFLAT_PER_WORKER, UFLAT_PER_WORKER)]
    )


def _sc_combine(xids, wids, v2d):
    mesh = plsc.VectorSubcoreMesh(core_axis_name="c", subcore_axis_name="s")
    fn = pl.kernel(
        _sc_combine_body,
        out_type=jax.ShapeDtypeStruct((N_BATCH * N_W * R,), jnp.float32),
        mesh=mesh,
        scratch_types=[
            pltpu.VMEM((LANES,), jnp.int32),
            pltpu.VMEM((LANES,), jnp.int32),
            pltpu.VMEM((LANES, N_W * R), jnp.float32),
            pltpu.VMEM((UFLAT_PER_WORKER,), jnp.float32),
            pltpu.SemaphoreType.DMA,
        ],
        compiler_params=pltpu.CompilerParams(needs_layout_passes=False),
    )
    return fn(xids, wids, v2d)


@jax.jit
def kernel(x, xids, wids, lora_A, lora_B):
    x2 = x.reshape(N_BATCH, D_IN).astype(jnp.bfloat16)
    a2 = lora_A.transpose(1, 0, 2).reshape(D_IN, N_W * R).astype(jnp.bfloat16)
    b2 = lora_B.reshape(N_W * R, D_OUT).astype(jnp.bfloat16)

    v = pl.pallas_call(
        _mm1_body,
        grid=(8,),
        in_specs=[
            pl.BlockSpec((N_BATCH, D_IN), lambda j: (0, 0)),
            pl.BlockSpec((D_IN, 256), lambda j: (0, j)),
        ],
        out_specs=pl.BlockSpec((N_BATCH, 256), lambda j: (0, j)),
        out_shape=jax.ShapeDtypeStruct((N_BATCH, N_W * R), jnp.float32),
    )(x2, a2)

    u_flat = _sc_combine(xids, wids, v)
    u2 = u_flat.reshape(N_BATCH, N_W * R)

    z = pl.pallas_call(
        _mm2_body,
        grid=(8,),
        in_specs=[
            pl.BlockSpec((N_BATCH, N_W * R), lambda j: (0, 0)),
            pl.BlockSpec((N_W * R, 512), lambda j: (0, j)),
        ],
        out_specs=pl.BlockSpec((N_BATCH, 512), lambda j: (0, j)),
        out_shape=jax.ShapeDtypeStruct((N_BATCH, D_OUT), jnp.float32),
        scratch_shapes=[pltpu.VMEM((N_BATCH, N_W * R), jnp.bfloat16)],
    )(u2, b2)

    return z.astype(jnp.float16).reshape(N_BATCH, 1, D_OUT)


# speedup vs baseline: 2.2150x; 1.0056x over previous
import functools

import jax
import jax.numpy as jnp
from jax import lax
from jax.experimental import pallas as pl
from jax.experimental.pallas import tpu as pltpu
from jax.experimental.pallas import tpu_sc as plsc

N_BATCH = 256
L_ROWS = 512
N_W = 128
D_IN = 4096
D_OUT = 4096
R = 16
LANES = 16
N_WORKERS = 32
ROWS_PER_WORKER = (N_BATCH * N_W) // N_WORKERS
UFLAT_PER_WORKER = ROWS_PER_WORKER * R


def _mm1_body(x_ref, a_ref, v_ref):
    v_ref[...] = jnp.dot(
        x_ref[...], a_ref[...], preferred_element_type=jnp.float32
    )


def _mm2_body(u_ref, b_ref, z_ref, uf16_ref):
    @pl.when(pl.program_id(0) == 0)
    def _():
        uf16_ref[...] = u_ref[...].astype(jnp.bfloat16)

    acc = jnp.dot(uf16_ref[...], b_ref[...], preferred_element_type=jnp.float32)
    z_ref[...] = acc * 2.0


def _sc_combine_body(xids_hbm, wids_hbm, v_hbm, out_hbm,
                     xv_ref, wv_ref, rows_ref, u_ref, sem):
    w = lax.axis_index("s") * 2 + lax.axis_index("c")
    iota = lax.iota(jnp.int32, LANES)

    def _zero(i, _):
        for u in range(8):
            u_ref[pl.ds((i * 8 + u) * LANES, LANES)] = jnp.zeros(
                (LANES,), jnp.float32
            )
        return 0

    lax.fori_loop(0, UFLAT_PER_WORKER // (LANES * 8), _zero, 0)

    def process_group(l_base, nvalid):
        pltpu.sync_copy(xids_hbm.at[pl.ds(l_base, LANES)], xv_ref)
        pltpu.sync_copy(wids_hbm.at[pl.ds(l_base, LANES)], wv_ref)
        xv = xv_ref[...]
        wv = wv_ref[...]
        lv = l_base + iota
        sv = jnp.where(
            lv < 128,
            lv,
            jnp.where(
                lv < 256,
                128 + ((lv - 128) >> 1),
                192 + ((lv - 256) >> 2),
            ),
        )
        pltpu.async_copy(v_hbm.at[xv], rows_ref, sem).wait()
        rloc = sv * N_W + wv - w * ROWS_PER_WORKER
        for j in range(LANES):
            jfull = jnp.full((LANES,), j, jnp.int32)
            mask = jfull < nvalid
            lane_j = iota == j
            wj = jnp.max(jnp.where(lane_j, wv, 0))
            rlj = jnp.max(jnp.where(lane_j, rloc, 0))
            offs = rlj * R + iota
            offs = jnp.where(mask, offs, 0)
            row = rows_ref[j, pl.ds(wj * R, R)]
            plsc.addupdate_scatter(u_ref, [offs], row, mask=mask)

    l1 = jnp.where(
        w < 16, 8 * w, jnp.where(w < 24, 16 * w - 128, 32 * w - 512)
    )
    n1 = jnp.where(w < 16, 8, 16)
    process_group(l1, n1)

    @pl.when(w >= 24)
    def _():
        process_group(32 * w - 512 + 16, jnp.int32(16))

    pltpu.sync_copy(
        u_ref, out_hbm.at[pl.ds(w * UFLAT_PER_WORKER, UFLAT_PER_WORKER)]
    )


def _sc_combine(xids, wids, v2d):
    mesh = plsc.VectorSubcoreMesh(core_axis_name="c", subcore_axis_name="s")
    fn = pl.kernel(
        _sc_combine_body,
        out_type=jax.ShapeDtypeStruct((N_BATCH * N_W * R,), jnp.float32),
        mesh=mesh,
        scratch_types=[
            pltpu.VMEM((LANES,), jnp.int32),
            pltpu.VMEM((LANES,), jnp.int32),
            pltpu.VMEM((LANES, N_W * R), jnp.float32),
            pltpu.VMEM((UFLAT_PER_WORKER,), jnp.float32),
            pltpu.SemaphoreType.DMA,
        ],
        compiler_params=pltpu.CompilerParams(needs_layout_passes=False),
    )
    return fn(xids, wids, v2d)


@jax.jit
def kernel(x, xids, wids, lora_A, lora_B):
    x2 = x.reshape(N_BATCH, D_IN).astype(jnp.bfloat16)
    a2 = lora_A.astype(jnp.bfloat16).transpose(1, 0, 2).reshape(D_IN, N_W * R)
    b2 = lora_B.reshape(N_W * R, D_OUT).astype(jnp.bfloat16)

    v = pl.pallas_call(
        _mm1_body,
        grid=(8,),
        in_specs=[
            pl.BlockSpec((N_BATCH, D_IN), lambda j: (0, 0)),
            pl.BlockSpec((D_IN, 256), lambda j: (0, j)),
        ],
        out_specs=pl.BlockSpec((N_BATCH, 256), lambda j: (0, j)),
        out_shape=jax.ShapeDtypeStruct((N_BATCH, N_W * R), jnp.float32),
    )(x2, a2)

    u_flat = _sc_combine(xids, wids, v)
    u2 = u_flat.reshape(N_BATCH, N_W * R)

    z = pl.pallas_call(
        _mm2_body,
        grid=(8,),
        in_specs=[
            pl.BlockSpec((N_BATCH, N_W * R), lambda j: (0, 0)),
            pl.BlockSpec((N_W * R, 512), lambda j: (0, j)),
        ],
        out_specs=pl.BlockSpec((N_BATCH, 512), lambda j: (0, j)),
        out_shape=jax.ShapeDtypeStruct((N_BATCH, D_OUT), jnp.float32),
        scratch_shapes=[pltpu.VMEM((N_BATCH, N_W * R), jnp.bfloat16)],
    )(u2, b2)

    return z.astype(jnp.float16).reshape(N_BATCH, 1, D_OUT)
